# baseline (speedup 1.0000x reference)
import functools

import jax
import jax.numpy as jnp
from jax import lax
from jax.experimental import pallas as pl
from jax.experimental.pallas import tpu as pltpu
from jax.experimental.pallas import tpu_sc as plsc

_L = 16


def _span_extract_sc(B, S, D, N, WPAD, n_workers):
    total = B * N
    assert total % n_workers == 0
    per_w = total // n_workers
    C = _L
    assert per_w % C == 0
    n_chunks = per_w // C
    NBUF = 2
    logN = N.bit_length() - 1
    assert N == (1 << logN)

    mesh = plsc.VectorSubcoreMesh(core_axis_name="c", subcore_axis_name="s")

    @functools.partial(
        pl.kernel,
        mesh=mesh,
        out_type=jax.ShapeDtypeStruct((total, 2 * D + WPAD), jnp.float32),
        scratch_types=[
            pltpu.VMEM((per_w,), jnp.int32),
            pltpu.VMEM((per_w,), jnp.int32),
            pltpu.VMEM((2 * per_w,), jnp.int32),
            pltpu.VMEM((per_w,), jnp.int32),
            pltpu.VMEM((NBUF, 2 * C, D), jnp.float32),
            pltpu.VMEM((NBUF, C, WPAD), jnp.float32),
            pltpu.SemaphoreType.DMA,
            pltpu.SemaphoreType.DMA,
        ],
    )
    def k(seq_hbm, starts_hbm, ends_hbm, wemb_hbm, out_hbm,
          rs_v, re_v, idx_v, widx_v, rows_v, wrow_v, gsem, wsem):
        wid = lax.axis_index("s") * 2 + lax.axis_index("c")
        base_span = wid * per_w
        lane = lax.iota(jnp.int32, _L)

        pltpu.sync_copy(starts_hbm.at[pl.ds(base_span, per_w)], rs_v)
        pltpu.sync_copy(ends_hbm.at[pl.ds(base_span, per_w)], re_v)
        for j in range(per_w // _L):
            off = j * _L + lane
            s = rs_v[pl.ds(j * _L, _L)]
            e = re_v[pl.ds(j * _L, _L)]
            fb = lax.shift_right_logical(base_span + off, logN) * S
            idx_v[pl.ds(2 * C * j, _L)] = fb + s
            idx_v[pl.ds(2 * C * j + C, _L)] = fb + e
            widx_v[pl.ds(j * _L, _L)] = e - s

        def fire_gather(c):
            slot = c % NBUF
            return (
                pltpu.async_copy(
                    seq_hbm.at[idx_v.at[pl.ds(2 * C * c, 2 * C)]],
                    rows_v.at[slot], gsem),
                pltpu.async_copy(
                    wemb_hbm.at[widx_v.at[pl.ds(C * c, C)]],
                    wrow_v.at[slot], gsem),
            )

        def fire_writes(c):
            slot = c % NBUF
            sb = base_span + c * C
            return (
                pltpu.async_copy(rows_v.at[slot, pl.ds(0, C)],
                                 out_hbm.at[pl.ds(sb, C), pl.ds(0, D)], wsem),
                pltpu.async_copy(rows_v.at[slot, pl.ds(C, C)],
                                 out_hbm.at[pl.ds(sb, C), pl.ds(D, D)], wsem),
                pltpu.async_copy(wrow_v.at[slot],
                                 out_hbm.at[pl.ds(sb, C), pl.ds(2 * D, WPAD)], wsem),
            )

        gathers = {0: fire_gather(0)}
        writes = {}
        for c in range(n_chunks):
            for d in gathers.pop(c):
                d.wait()
            writes[c] = fire_writes(c)
            if c + 1 < n_chunks:
                prev = c + 1 - NBUF
                if prev >= 0:
                    for d in writes.pop(prev):
                        d.wait()
                gathers[c + 1] = fire_gather(c + 1)
        for ds in writes.values():
            for d in ds:
                d.wait()

    return k


def _assemble_tc(B, N, D, WDIM, WPAD):
    F = 2 * D + WDIM
    n_fblocks = (F + WPAD - 1) // WPAD

    def body(x_ref, o_ref):
        o_ref[...] = jnp.transpose(x_ref[...])[None]

    return pl.pallas_call(
        body,
        grid=(B, n_fblocks),
        in_specs=[pl.BlockSpec((N, WPAD), lambda b, f: (b, f))],
        out_specs=pl.BlockSpec((1, WPAD, N), lambda b, f: (b, f, 0)),
        out_shape=jax.ShapeDtypeStruct((B, F, N), jnp.float32),
    )


def kernel(sequence_tensor, span_indices, width_embedding):
    B, S, D = sequence_tensor.shape
    _, N, _ = span_indices.shape
    WDIM = width_embedding.shape[1]
    seq_flat = sequence_tensor.reshape(B * S, D)
    starts_flat = span_indices[:, :, 0].reshape(-1).astype(jnp.int32)
    ends_flat = span_indices[:, :, 1].reshape(-1).astype(jnp.int32)
    WPAD = ((WDIM + 127) // 128) * 128
    wemb = jnp.pad(width_embedding, ((0, 0), (0, WPAD - WDIM)))
    rows = _span_extract_sc(B, S, D, N, WPAD, 32)(
        seq_flat, starts_flat, ends_flat, wemb)
    rows = pltpu.with_memory_space_constraint(rows, pltpu.MemorySpace.HBM)
    out_t = _assemble_tc(B, N, D, WDIM, WPAD)(rows)
    return jnp.transpose(out_t, (0, 2, 1))


# speedup vs baseline: 1.5093x; 1.5093x over previous
import functools

import jax
import jax.numpy as jnp
from jax import lax
from jax.experimental import pallas as pl
from jax.experimental.pallas import tpu as pltpu
from jax.experimental.pallas import tpu_sc as plsc

_L = 16


def _span_extract_sc(B, S, D, N, WDIM, WPAD, n_workers):
    total = B * N
    assert total % n_workers == 0
    per_w = total // n_workers
    C = _L
    assert per_w % C == 0
    n_chunks = per_w // C
    NBUF = 3
    logN = N.bit_length() - 1
    assert N == (1 << logN)

    mesh = plsc.VectorSubcoreMesh(core_axis_name="c", subcore_axis_name="s")

    @functools.partial(
        pl.kernel,
        mesh=mesh,
        out_type=(
            jax.ShapeDtypeStruct((total, 2 * D + WDIM), jnp.float32),
            jax.ShapeDtypeStruct((total, WPAD), jnp.float32),
        ),
        scratch_types=[
            pltpu.VMEM((per_w,), jnp.int32),
            pltpu.VMEM((per_w,), jnp.int32),
            pltpu.VMEM((2 * per_w,), jnp.int32),
            pltpu.VMEM((per_w,), jnp.int32),
            pltpu.VMEM((NBUF, 2 * C, D), jnp.float32),
            pltpu.VMEM((NBUF, C, WPAD), jnp.float32),
            pltpu.SemaphoreType.DMA,
            pltpu.SemaphoreType.DMA,
        ],
    )
    def k(seq_hbm, starts_hbm, ends_hbm, wemb_hbm, out_hbm, outw_hbm,
          rs_v, re_v, idx_v, widx_v, rows_v, wrow_v, gsem, wsem):
        wid = lax.axis_index("s") * 2 + lax.axis_index("c")
        base_span = wid * per_w
        lane = lax.iota(jnp.int32, _L)

        pltpu.sync_copy(starts_hbm.at[pl.ds(base_span, per_w)], rs_v)
        pltpu.sync_copy(ends_hbm.at[pl.ds(base_span, per_w)], re_v)
        for j in range(per_w // _L):
            off = j * _L + lane
            s = rs_v[pl.ds(j * _L, _L)]
            e = re_v[pl.ds(j * _L, _L)]
            fb = lax.shift_right_logical(base_span + off, logN) * S
            idx_v[pl.ds(2 * C * j, _L)] = fb + s
            idx_v[pl.ds(2 * C * j + C, _L)] = fb + e
            widx_v[pl.ds(j * _L, _L)] = e - s

        def fire_gather(c):
            slot = c % NBUF
            return (
                pltpu.async_copy(
                    seq_hbm.at[idx_v.at[pl.ds(2 * C * c, 2 * C)]],
                    rows_v.at[slot], gsem),
                pltpu.async_copy(
                    wemb_hbm.at[widx_v.at[pl.ds(C * c, C)]],
                    wrow_v.at[slot], gsem),
            )

        def fire_writes(c):
            slot = c % NBUF
            sb = base_span + c * C
            return (
                pltpu.async_copy(rows_v.at[slot, pl.ds(0, C)],
                                 out_hbm.at[pl.ds(sb, C), pl.ds(0, D)], wsem),
                pltpu.async_copy(rows_v.at[slot, pl.ds(C, C)],
                                 out_hbm.at[pl.ds(sb, C), pl.ds(D, D)], wsem),
                pltpu.async_copy(wrow_v.at[slot],
                                 outw_hbm.at[pl.ds(sb, C)], wsem),
            )

        gathers = {}
        writes = {}
        for c in range(min(NBUF - 1, n_chunks)):
            gathers[c] = fire_gather(c)
        for c in range(n_chunks):
            for d in gathers.pop(c):
                d.wait()
            writes[c] = fire_writes(c)
            nxt = c + NBUF - 1
            if nxt < n_chunks:
                prev = nxt - NBUF
                if prev >= 0:
                    for d in writes.pop(prev):
                        d.wait()
                gathers[nxt] = fire_gather(nxt)
        for ds in writes.values():
            for d in ds:
                d.wait()

    return k


def _width_fill_tc(total, D, WDIM, WPAD, R):
    def body(w_ref, main_ref, out_ref):
        del main_ref
        out_ref[...] = w_ref[...]

    return pl.pallas_call(
        body,
        grid=(total // R,),
        in_specs=[
            pl.BlockSpec((R, WPAD), lambda i: (i, 0)),
            pl.BlockSpec(memory_space=pltpu.MemorySpace.HBM),
        ],
        out_specs=pl.BlockSpec((R, WPAD), lambda i: (i, (2 * D) // WPAD)),
        out_shape=jax.ShapeDtypeStruct((total, 2 * D + WDIM), jnp.float32),
        input_output_aliases={1: 0},
    )


def kernel(sequence_tensor, span_indices, width_embedding):
    B, S, D = sequence_tensor.shape
    _, N, _ = span_indices.shape
    WDIM = width_embedding.shape[1]
    total = B * N
    seq_flat = sequence_tensor.reshape(B * S, D)
    starts_flat = span_indices[:, :, 0].reshape(-1).astype(jnp.int32)
    ends_flat = span_indices[:, :, 1].reshape(-1).astype(jnp.int32)
    WPAD = ((WDIM + 127) // 128) * 128
    wemb = jnp.pad(width_embedding, ((0, 0), (0, WPAD - WDIM)))
    out_main, out_w = _span_extract_sc(B, S, D, N, WDIM, WPAD, 32)(
        seq_flat, starts_flat, ends_flat, wemb)
    out_main = pltpu.with_memory_space_constraint(out_main, pltpu.MemorySpace.HBM)
    out_w = pltpu.with_memory_space_constraint(out_w, pltpu.MemorySpace.HBM)
    out = _width_fill_tc(total, D, WDIM, WPAD, 512)(out_w, out_main)
    return out.reshape(B, N, 2 * D + WDIM)


# speedup vs baseline: 1.5594x; 1.0332x over previous
import functools

import jax
import jax.numpy as jnp
from jax import lax
from jax.experimental import pallas as pl
from jax.experimental.pallas import tpu as pltpu
from jax.experimental.pallas import tpu_sc as plsc

_L = 16


def _span_extract_sc(B, S, D, N, WDIM, WPAD, n_workers):
    total = B * N
    assert total % n_workers == 0
    per_w = total // n_workers
    C = _L
    assert per_w % C == 0
    n_chunks = per_w // C
    NBUF = 3
    logN = N.bit_length() - 1
    assert N == (1 << logN)

    mesh = plsc.VectorSubcoreMesh(core_axis_name="c", subcore_axis_name="s")

    @functools.partial(
        pl.kernel,
        mesh=mesh,
        out_type=jax.ShapeDtypeStruct((total, 2 * D + WPAD), jnp.float32),
        scratch_types=[
            pltpu.VMEM((per_w,), jnp.int32),
            pltpu.VMEM((per_w,), jnp.int32),
            pltpu.VMEM((2 * per_w,), jnp.int32),
            pltpu.VMEM((per_w,), jnp.int32),
            pltpu.VMEM((NBUF, 2 * C, D), jnp.float32),
            pltpu.VMEM((NBUF, C, WPAD), jnp.float32),
            pltpu.SemaphoreType.DMA,
            pltpu.SemaphoreType.DMA,
        ],
    )
    def k(seq_hbm, starts_hbm, ends_hbm, wemb_hbm, out_hbm,
          rs_v, re_v, idx_v, widx_v, rows_v, wrow_v, gsem, wsem):
        wid = lax.axis_index("s") * 2 + lax.axis_index("c")
        base_span = wid * per_w
        lane = lax.iota(jnp.int32, _L)

        pltpu.sync_copy(starts_hbm.at[pl.ds(base_span, per_w)], rs_v)
        pltpu.sync_copy(ends_hbm.at[pl.ds(base_span, per_w)], re_v)
        for j in range(per_w // _L):
            off = j * _L + lane
            s = rs_v[pl.ds(j * _L, _L)]
            e = re_v[pl.ds(j * _L, _L)]
            fb = lax.shift_right_logical(base_span + off, logN) * S
            idx_v[pl.ds(2 * C * j, _L)] = fb + s
            idx_v[pl.ds(2 * C * j + C, _L)] = fb + e
            widx_v[pl.ds(j * _L, _L)] = e - s

        def fire_gather(c):
            slot = c % NBUF
            return (
                pltpu.async_copy(
                    seq_hbm.at[idx_v.at[pl.ds(2 * C * c, 2 * C)]],
                    rows_v.at[slot], gsem),
                pltpu.async_copy(
                    wemb_hbm.at[widx_v.at[pl.ds(C * c, C)]],
                    wrow_v.at[slot], gsem),
            )

        def fire_writes(c):
            slot = c % NBUF
            sb = base_span + c * C
            return (
                pltpu.async_copy(rows_v.at[slot, pl.ds(0, C)],
                                 out_hbm.at[pl.ds(sb, C), pl.ds(0, D)], wsem),
                pltpu.async_copy(rows_v.at[slot, pl.ds(C, C)],
                                 out_hbm.at[pl.ds(sb, C), pl.ds(D, D)], wsem),
                pltpu.async_copy(wrow_v.at[slot],
                                 out_hbm.at[pl.ds(sb, C), pl.ds(2 * D, WPAD)], wsem),
            )

        gathers = {}
        writes = {}
        for c in range(min(NBUF - 1, n_chunks)):
            gathers[c] = fire_gather(c)
        for c in range(n_chunks):
            for d in gathers.pop(c):
                d.wait()
            writes[c] = fire_writes(c)
            nxt = c + NBUF - 1
            if nxt < n_chunks:
                prev = nxt - NBUF
                if prev >= 0:
                    for d in writes.pop(prev):
                        d.wait()
                gathers[nxt] = fire_gather(nxt)
        for ds in writes.values():
            for d in ds:
                d.wait()

    return k


def kernel(sequence_tensor, span_indices, width_embedding):
    B, S, D = sequence_tensor.shape
    _, N, _ = span_indices.shape
    WDIM = width_embedding.shape[1]
    total = B * N
    seq_flat = sequence_tensor.reshape(B * S, D)
    starts_flat = span_indices[:, :, 0].reshape(-1).astype(jnp.int32)
    ends_flat = span_indices[:, :, 1].reshape(-1).astype(jnp.int32)
    WPAD = ((WDIM + 127) // 128) * 128
    wemb = jnp.pad(width_embedding, ((0, 0), (0, WPAD - WDIM)))
    rows = _span_extract_sc(B, S, D, N, WDIM, WPAD, 32)(
        seq_flat, starts_flat, ends_flat, wemb)
    rows = pltpu.with_memory_space_constraint(rows, pltpu.MemorySpace.HBM)
    return rows[:, :2 * D + WDIM].reshape(B, N, 2 * D + WDIM)


# speedup vs baseline: 1.5699x; 1.0067x over previous
import functools

import jax
import jax.numpy as jnp
from jax import lax
from jax.experimental import pallas as pl
from jax.experimental.pallas import tpu as pltpu
from jax.experimental.pallas import tpu_sc as plsc

_L = 16


def _span_extract_sc(B, S, D, N, WDIM, WPAD, n_workers):
    total = B * N
    assert total % n_workers == 0
    per_w = total // n_workers
    C = _L
    assert per_w % C == 0
    n_chunks = per_w // C
    NBUF = 3
    logN = N.bit_length() - 1
    assert N == (1 << logN)

    mesh = plsc.VectorSubcoreMesh(core_axis_name="c", subcore_axis_name="s")

    @functools.partial(
        pl.kernel,
        mesh=mesh,
        out_type=jax.ShapeDtypeStruct((total, 2 * D + WPAD), jnp.float32),
        scratch_types=[
            pltpu.VMEM((per_w,), jnp.int32),
            pltpu.VMEM((per_w,), jnp.int32),
            pltpu.VMEM((2 * per_w,), jnp.int32),
            pltpu.VMEM((per_w,), jnp.int32),
            pltpu.VMEM((NBUF, 2 * C, D), jnp.float32),
            pltpu.VMEM((NBUF, C, WPAD), jnp.float32),
            pltpu.SemaphoreType.DMA,
            pltpu.SemaphoreType.DMA,
        ],
    )
    def k(seq_hbm, starts_hbm, ends_hbm, wemb_hbm, out_hbm,
          rs_v, re_v, idx_v, widx_v, rows_v, wrow_v, gsem, wsem):
        wid = lax.axis_index("s") * 2 + lax.axis_index("c")
        base_span = wid * per_w
        lane = lax.iota(jnp.int32, _L)

        c1 = pltpu.async_copy(starts_hbm.at[pl.ds(base_span, per_w)], rs_v, gsem)
        c2 = pltpu.async_copy(ends_hbm.at[pl.ds(base_span, per_w)], re_v, gsem)
        c1.wait()
        c2.wait()

        def compute_indices(j):
            off = j * _L + lane
            s = rs_v[pl.ds(j * _L, _L)]
            e = re_v[pl.ds(j * _L, _L)]
            fb = lax.shift_right_logical(base_span + off, logN) * S
            idx_v[pl.ds(2 * C * j, _L)] = fb + s
            idx_v[pl.ds(2 * C * j + C, _L)] = fb + e
            widx_v[pl.ds(j * _L, _L)] = e - s

        def fire_gather(c):
            slot = c % NBUF
            return (
                pltpu.async_copy(
                    seq_hbm.at[idx_v.at[pl.ds(2 * C * c, 2 * C)]],
                    rows_v.at[slot], gsem),
                pltpu.async_copy(
                    wemb_hbm.at[widx_v.at[pl.ds(C * c, C)]],
                    wrow_v.at[slot], gsem),
            )

        def fire_writes(c):
            slot = c % NBUF
            sb = base_span + c * C
            return (
                pltpu.async_copy(rows_v.at[slot, pl.ds(0, C)],
                                 out_hbm.at[pl.ds(sb, C), pl.ds(0, D)], wsem),
                pltpu.async_copy(rows_v.at[slot, pl.ds(C, C)],
                                 out_hbm.at[pl.ds(sb, C), pl.ds(D, D)], wsem),
                pltpu.async_copy(wrow_v.at[slot],
                                 out_hbm.at[pl.ds(sb, C), pl.ds(2 * D, WPAD)], wsem),
            )

        gathers = {}
        writes = {}
        n_prime = min(NBUF - 1, n_chunks)
        assert C == _L
        for c in range(n_prime):
            compute_indices(c)
            gathers[c] = fire_gather(c)
        for j in range(n_prime, per_w // _L):
            compute_indices(j)
        for c in range(n_chunks):
            for d in gathers.pop(c):
                d.wait()
            writes[c] = fire_writes(c)
            nxt = c + NBUF - 1
            if nxt < n_chunks:
                prev = nxt - NBUF
                if prev >= 0:
                    for d in writes.pop(prev):
                        d.wait()
                gathers[nxt] = fire_gather(nxt)
        for ds in writes.values():
            for d in ds:
                d.wait()

    return k


def kernel(sequence_tensor, span_indices, width_embedding):
    B, S, D = sequence_tensor.shape
    _, N, _ = span_indices.shape
    WDIM = width_embedding.shape[1]
    total = B * N
    seq_flat = sequence_tensor.reshape(B * S, D)
    starts_flat = span_indices[:, :, 0].reshape(-1).astype(jnp.int32)
    ends_flat = span_indices[:, :, 1].reshape(-1).astype(jnp.int32)
    WPAD = ((WDIM + 127) // 128) * 128
    wemb = jnp.pad(width_embedding, ((0, 0), (0, WPAD - WDIM)))
    rows = _span_extract_sc(B, S, D, N, WDIM, WPAD, 32)(
        seq_flat, starts_flat, ends_flat, wemb)
    rows = pltpu.with_memory_space_constraint(rows, pltpu.MemorySpace.HBM)
    return rows[:, :2 * D + WDIM].reshape(B, N, 2 * D + WDIM)


# speedup vs baseline: 1.5889x; 1.0121x over previous
import functools

import jax
import jax.numpy as jnp
from jax import lax
from jax.experimental import pallas as pl
from jax.experimental.pallas import tpu as pltpu
from jax.experimental.pallas import tpu_sc as plsc

_L = 16


def _span_extract_sc(B, S, D, N, WDIM, WPAD, n_workers):
    total = B * N
    assert total % n_workers == 0
    per_w = total // n_workers
    C = _L
    assert per_w % C == 0
    n_chunks = per_w // C
    NBUF = 3
    logN = N.bit_length() - 1
    assert N == (1 << logN)

    mesh = plsc.VectorSubcoreMesh(core_axis_name="c", subcore_axis_name="s")

    @functools.partial(
        pl.kernel,
        mesh=mesh,
        out_type=jax.ShapeDtypeStruct((total, 2 * D + WPAD), jnp.float32),
        scratch_types=[
            pltpu.VMEM((per_w,), jnp.int32),
            pltpu.VMEM((per_w,), jnp.int32),
            pltpu.VMEM((2 * per_w,), jnp.int32),
            pltpu.VMEM((per_w,), jnp.int32),
            pltpu.VMEM((NBUF, 2 * C, D), jnp.float32),
            pltpu.VMEM((per_w, WPAD), jnp.float32),
            pltpu.SemaphoreType.DMA,
            pltpu.SemaphoreType.DMA,
            pltpu.SemaphoreType.DMA,
        ],
    )
    def k(seq_hbm, starts_hbm, ends_hbm, wemb_hbm, out_hbm,
          rs_v, re_v, idx_v, widx_v, rows_v, wrow_v, gsem, wsem, wsem2):
        wid = lax.axis_index("s") * 2 + lax.axis_index("c")
        base_span = wid * per_w
        lane = lax.iota(jnp.int32, _L)

        c1 = pltpu.async_copy(starts_hbm.at[pl.ds(base_span, per_w)], rs_v, gsem)
        c2 = pltpu.async_copy(ends_hbm.at[pl.ds(base_span, per_w)], re_v, gsem)
        c1.wait()
        c2.wait()

        def compute_indices(j):
            off = j * _L + lane
            s = rs_v[pl.ds(j * _L, _L)]
            e = re_v[pl.ds(j * _L, _L)]
            fb = lax.shift_right_logical(base_span + off, logN) * S
            idx_v[pl.ds(2 * C * j, _L)] = fb + s
            idx_v[pl.ds(2 * C * j + C, _L)] = fb + e
            widx_v[pl.ds(j * _L, _L)] = e - s

        def fire_gather(c):
            slot = c % NBUF
            return (
                pltpu.async_copy(
                    seq_hbm.at[idx_v.at[pl.ds(2 * C * c, 2 * C)]],
                    rows_v.at[slot], gsem),
            )

        def fire_writes(c):
            slot = c % NBUF
            sb = base_span + c * C
            return (
                pltpu.async_copy(rows_v.at[slot, pl.ds(0, C)],
                                 out_hbm.at[pl.ds(sb, C), pl.ds(0, D)], wsem),
                pltpu.async_copy(rows_v.at[slot, pl.ds(C, C)],
                                 out_hbm.at[pl.ds(sb, C), pl.ds(D, D)], wsem),
            )

        gathers = {}
        writes = {}
        n_prime = min(NBUF - 1, n_chunks)
        assert C == _L
        for c in range(n_prime):
            compute_indices(c)
            gathers[c] = fire_gather(c)
        for j in range(n_prime, per_w // _L):
            compute_indices(j)
        wg = pltpu.async_copy(wemb_hbm.at[widx_v], wrow_v, wsem2)
        wg.wait()
        wwrite = pltpu.async_copy(
            wrow_v, out_hbm.at[pl.ds(base_span, per_w), pl.ds(2 * D, WPAD)], wsem2)
        for c in range(n_chunks):
            for d in gathers.pop(c):
                d.wait()
            writes[c] = fire_writes(c)
            nxt = c + NBUF - 1
            if nxt < n_chunks:
                prev = nxt - NBUF
                if prev >= 0:
                    for d in writes.pop(prev):
                        d.wait()
                gathers[nxt] = fire_gather(nxt)
        for ds in writes.values():
            for d in ds:
                d.wait()
        wwrite.wait()

    return k


def kernel(sequence_tensor, span_indices, width_embedding):
    B, S, D = sequence_tensor.shape
    _, N, _ = span_indices.shape
    WDIM = width_embedding.shape[1]
    total = B * N
    seq_flat = sequence_tensor.reshape(B * S, D)
    starts_flat = span_indices[:, :, 0].reshape(-1).astype(jnp.int32)
    ends_flat = span_indices[:, :, 1].reshape(-1).astype(jnp.int32)
    WPAD = ((WDIM + 127) // 128) * 128
    wemb = jnp.pad(width_embedding, ((0, 0), (0, WPAD - WDIM)))
    rows = _span_extract_sc(B, S, D, N, WDIM, WPAD, 32)(
        seq_flat, starts_flat, ends_flat, wemb)
    return rows[:, :2 * D + WDIM].reshape(B, N, 2 * D + WDIM)
